# baseline (speedup 1.0000x reference)
import jax
import jax.numpy as jnp
from jax import lax
from jax.experimental import pallas as pl
from jax.experimental.pallas import tpu as pltpu
from jax.experimental.pallas import tpu_sc as plsc

B, H, Q, DH = 16, 8, 16, 128
MAX_LEN = 4096
N_POSITIONS = 2048

NUM_WORKERS = 32
TOTAL_ROWS = B * H * MAX_LEN
ROWS_PER_WORKER = TOTAL_ROWS // NUM_WORKERS
GROUPS_PER_WORKER = ROWS_PER_WORKER // MAX_LEN
SCATTER_ROWS = GROUPS_PER_WORKER * Q
LANES = 16
CHUNKS_PER_ROW = DH // LANES

COPY_CHUNK = 256
N_CHUNKS = ROWS_PER_WORKER // COPY_CHUNK

TC_CHUNK = 1024
TC_N_CHUNKS = MAX_LEN // TC_CHUNK


def _copy_stream(src, dst, row0, b0, b1, sr0, sr1, sw0, sw1):
    def rd(c, buf, sem):
        return pltpu.make_async_copy(
            src.at[pl.ds(row0 + c * COPY_CHUNK, COPY_CHUNK)], buf, sem)

    def wr(c, buf, sem):
        return pltpu.make_async_copy(
            buf, dst.at[pl.ds(row0 + c * COPY_CHUNK, COPY_CHUNK)], sem)

    rd(0, b0, sr0).start()

    def body(p, carry):
        c0 = 2 * p

        @pl.when(p > 0)
        def _():
            wr(c0 - 1, b1, sw1).wait()

        rd(c0 + 1, b1, sr1).start()
        rd(c0, b0, sr0).wait()
        wr(c0, b0, sw0).start()

        @pl.when(p < N_CHUNKS // 2 - 1)
        def _():
            wr(c0, b0, sw0).wait()
            rd(c0 + 2, b0, sr0).start()

        rd(c0 + 1, b1, sr1).wait()
        wr(c0 + 1, b1, sw1).start()
        return carry

    lax.fori_loop(0, N_CHUNKS // 2, body, 0)
    wr(N_CHUNKS - 2, b0, sw0).wait()
    wr(N_CHUNKS - 1, b1, sw1).wait()


def _sc_body(vs, pos, vc, vo,
             pos_v, ext, sidx_l, sidx_f, didx, vrl, vrf,
             cb0, cb1,
             sem_r0, sem_r1, sem_w0, sem_w1, sem_gvl, sem_gvf, sem_sv):
    c = lax.axis_index("c")
    s = lax.axis_index("s")
    wid = s * 2 + c
    row0 = wid * ROWS_PER_WORKER
    g0 = wid * GROUPS_PER_WORKER
    b = g0 // H

    pltpu.sync_copy(pos.at[pl.ds(b * Q, Q)], pos_v)
    pv = pos_v[...]
    iota = lax.iota(jnp.int32, Q)
    ext[pl.ds(Q, Q)] = jnp.full((Q,), -1, jnp.int32)
    ext[pl.ds(0, Q)] = pv
    nxt = ext[pl.ds(1, Q)]
    ext[pl.ds(0, Q)] = jnp.full((Q,), -1, jnp.int32)
    ext[pl.ds(Q, Q)] = pv
    prv = ext[pl.ds(Q - 1, Q)]
    xl = jnp.where(pv != nxt, iota, Q - 1)
    for k in (1, 2, 4, 8):
        ext[pl.ds(Q, Q)] = jnp.full((Q,), Q - 1, jnp.int32)
        ext[pl.ds(0, Q)] = xl
        xl = jnp.minimum(xl, ext[pl.ds(k, Q)])
    xf = jnp.where(pv != prv, iota, 0)
    for k in (1, 2, 4, 8):
        ext[pl.ds(0, Q)] = jnp.zeros((Q,), jnp.int32)
        ext[pl.ds(Q, Q)] = xf
        xf = jnp.maximum(xf, ext[pl.ds(Q - k, Q)])
    for j in range(GROUPS_PER_WORKER):
        g = g0 + j
        sidx_l[pl.ds(j * Q, Q)] = g * Q + xl
        sidx_f[pl.ds(j * Q, Q)] = g * Q + xf
        didx[pl.ds(j * Q, Q)] = g * MAX_LEN + pv

    g_vl = pltpu.make_async_copy(vs.at[sidx_l], vrl, sem_gvl)
    g_vf = pltpu.make_async_copy(vs.at[sidx_f], vrf, sem_gvf)
    g_vl.start()
    g_vf.start()
    g_vl.wait()
    g_vf.wait()

    lane_par = lax.iota(jnp.int32, LANES) & 1

    def _blend_body(t, carry):
        r = t // CHUNKS_PER_ROW
        ch = t % CHUNKS_PER_ROW
        off = ch * LANES
        want = jnp.where(ch < CHUNKS_PER_ROW // 2, 1, 0)
        m = lane_par == want
        vrl[r, pl.ds(off, LANES)] = jnp.where(m, vrf[r, pl.ds(off, LANES)],
                                              vrl[r, pl.ds(off, LANES)])
        return carry

    lax.fori_loop(0, SCATTER_ROWS * CHUNKS_PER_ROW, _blend_body, 0)

    _copy_stream(vc, vo, row0, cb0, cb1, sem_r0, sem_r1, sem_w0, sem_w1)
    s_v = pltpu.make_async_copy(vrl, vo.at[didx], sem_sv)
    s_v.start()
    s_v.wait()


def _sc_cache_update(vs_flat, pos_flat, vc_flat):
    mesh = plsc.VectorSubcoreMesh(core_axis_name="c", subcore_axis_name="s")
    run = pl.kernel(
        _sc_body,
        out_type=jax.ShapeDtypeStruct((TOTAL_ROWS, DH), jnp.float32),
        mesh=mesh,
        scratch_types=[
            pltpu.VMEM((Q,), jnp.int32),
            pltpu.VMEM((2 * Q,), jnp.int32),
            pltpu.VMEM((SCATTER_ROWS,), jnp.int32),
            pltpu.VMEM((SCATTER_ROWS,), jnp.int32),
            pltpu.VMEM((SCATTER_ROWS,), jnp.int32),
            pltpu.VMEM((SCATTER_ROWS, DH), jnp.float32),
            pltpu.VMEM((SCATTER_ROWS, DH), jnp.float32),
            pltpu.VMEM((COPY_CHUNK, DH), jnp.float32),
            pltpu.VMEM((COPY_CHUNK, DH), jnp.float32),
        ] + [pltpu.SemaphoreType.DMA] * 7,
    )
    return run(vs_flat, pos_flat, vc_flat)


def _tc_body(pos_ref, fidx_ref, lidx_ref, ks_ref, kc_ref, ko_ref):
    b = pl.program_id(0)
    c = pl.program_id(2)
    ko_ref[...] = kc_ref[...]
    base = c * TC_CHUNK
    lane = lax.broadcasted_iota(jnp.int32, (1, DH), 1)
    m_last = (lane & 1) == jnp.where(lane >= DH // 2, 1, 0)

    @pl.when(base < N_POSITIONS)
    def _scatter():
        for q in range(Q):
            p = pos_ref[b, q]

            @pl.when((p >= base) & (p < base + TC_CHUNK))
            def _write():
                fq = fidx_ref[b, q]
                lq = lidx_ref[b, q]
                rowf = ks_ref[0, 0, pl.ds(fq, 1), :]
                rowl = ks_ref[0, 0, pl.ds(lq, 1), :]
                ko_ref[0, 0, pl.ds(p - base, 1), :] = jnp.where(m_last, rowl, rowf)


def _tc_cache_update(key_states, position_ids, fidx, lidx, k_cache):
    grid = (B, H, TC_N_CHUNKS)
    states_spec = pl.BlockSpec((1, 1, Q, DH), lambda b, h, c, *_: (b, h, 0, 0))
    cache_spec = pl.BlockSpec((1, 1, TC_CHUNK, DH), lambda b, h, c, *_: (b, h, c, 0))
    grid_spec = pltpu.PrefetchScalarGridSpec(
        num_scalar_prefetch=3,
        grid=grid,
        in_specs=[states_spec, cache_spec],
        out_specs=cache_spec,
    )
    return pl.pallas_call(
        _tc_body,
        grid_spec=grid_spec,
        out_shape=jax.ShapeDtypeStruct((B, H, MAX_LEN, DH), jnp.float32),
        compiler_params=pltpu.CompilerParams(
            dimension_semantics=("parallel", "parallel", "arbitrary"),
        ),
    )(position_ids, fidx, lidx, key_states, k_cache)


@jax.jit
def _cache_update(key_states, value_states, position_ids, k_cache, v_cache):
    pos = position_ids.astype(jnp.int32)
    iota = jnp.arange(Q, dtype=jnp.int32)[None, :]
    nxt = jnp.concatenate([pos[:, 1:], jnp.full((B, 1), -1, jnp.int32)], axis=1)
    prv = jnp.concatenate([jnp.full((B, 1), -1, jnp.int32), pos[:, :-1]], axis=1)
    xl = jnp.where(pos != nxt, iota, Q - 1)
    xl = lax.rev(lax.cummin(lax.rev(xl, (1,)), axis=1), (1,))
    xf = jnp.where(pos != prv, iota, 0)
    xf = lax.cummax(xf, axis=1)

    v_out = _sc_cache_update(
        value_states.reshape(B * H * Q, DH),
        pos.reshape(B * Q),
        v_cache.reshape(TOTAL_ROWS, DH),
    ).reshape(B, H, MAX_LEN, DH)
    k_out = _tc_cache_update(key_states, pos, xf, xl, k_cache)
    return k_out, v_out


def kernel(key_states, value_states, position_ids, k_cache, v_cache, n_positions):
    return tuple(_cache_update(key_states, value_states, position_ids,
                               k_cache, v_cache))


# speedup vs baseline: 1.2907x; 1.2907x over previous
import functools

import jax
import jax.numpy as jnp
from jax import lax
from jax.experimental import pallas as pl
from jax.experimental.pallas import tpu as pltpu
from jax.experimental.pallas import tpu_sc as plsc

B, H, Q, DH = 16, 8, 16, 128
MAX_LEN = 4096
N_POSITIONS = 2048

NUM_WORKERS = 32
TOTAL_ROWS = B * H * MAX_LEN
ROWS_PER_WORKER = TOTAL_ROWS // NUM_WORKERS
GROUPS_PER_WORKER = ROWS_PER_WORKER // MAX_LEN
SCATTER_ROWS = GROUPS_PER_WORKER * Q
LANES = 16
CHUNKS_PER_ROW = DH // LANES


COPY_CHUNK = 256
N_CHUNKS = ROWS_PER_WORKER // COPY_CHUNK


def _copy_stream(src, dst, row0, b0, b1, sr0, sr1, sw0, sw1):
    def rd(c, buf, sem):
        return pltpu.make_async_copy(
            src.at[pl.ds(row0 + c * COPY_CHUNK, COPY_CHUNK)], buf, sem)

    def wr(c, buf, sem):
        return pltpu.make_async_copy(
            buf, dst.at[pl.ds(row0 + c * COPY_CHUNK, COPY_CHUNK)], sem)

    rd(0, b0, sr0).start()

    def body(p, carry):
        c0 = 2 * p

        @pl.when(p > 0)
        def _():
            wr(c0 - 1, b1, sw1).wait()

        rd(c0 + 1, b1, sr1).start()
        rd(c0, b0, sr0).wait()
        wr(c0, b0, sw0).start()

        @pl.when(p < N_CHUNKS // 2 - 1)
        def _():
            wr(c0, b0, sw0).wait()
            rd(c0 + 2, b0, sr0).start()

        rd(c0 + 1, b1, sr1).wait()
        wr(c0 + 1, b1, sw1).start()
        return carry

    lax.fori_loop(0, N_CHUNKS // 2, body, 0)
    wr(N_CHUNKS - 2, b0, sw0).wait()
    wr(N_CHUNKS - 1, b1, sw1).wait()


def _sc_body(ks, vs, pos, mmap, kc, vc, ko, vo,
             pos_v, ext, sidx_l, sidx_f, sidx_m, didx,
             krl, krf, krm, vrl, vrf, vrm, mrows,
             cb0, cb1,
             sem_r0, sem_r1, sem_w0, sem_w1,
             sem_gkl, sem_gkf, sem_gkm, sem_gvl, sem_gvf, sem_gvm,
             sem_sk, sem_sv):
    c = lax.axis_index("c")
    s = lax.axis_index("s")
    wid = s * 2 + c
    row0 = wid * ROWS_PER_WORKER
    g0 = wid * GROUPS_PER_WORKER
    b = g0 // H

    pltpu.sync_copy(pos.at[pl.ds(b * Q, Q)], pos_v)
    pv = pos_v[...]
    iota = lax.iota(jnp.int32, Q)
    ext[pl.ds(Q, Q)] = jnp.full((Q,), -1, jnp.int32)
    ext[pl.ds(0, Q)] = pv
    nxt = ext[pl.ds(1, Q)]
    ext[pl.ds(0, Q)] = jnp.full((Q,), -1, jnp.int32)
    ext[pl.ds(Q, Q)] = pv
    prv = ext[pl.ds(Q - 1, Q)]
    xl = jnp.where(pv != nxt, iota, Q - 1)
    for k in (1, 2, 4, 8):
        ext[pl.ds(Q, Q)] = jnp.full((Q,), Q - 1, jnp.int32)
        ext[pl.ds(0, Q)] = xl
        xl = jnp.minimum(xl, ext[pl.ds(k, Q)])
    xf = jnp.where(pv != prv, iota, 0)
    for k in (1, 2, 4, 8):
        ext[pl.ds(0, Q)] = jnp.zeros((Q,), jnp.int32)
        ext[pl.ds(Q, Q)] = xf
        xf = jnp.maximum(xf, ext[pl.ds(Q - k, Q)])
    xm = jnp.minimum(xf + 1, xl)
    for j in range(GROUPS_PER_WORKER):
        g = g0 + j
        sidx_l[pl.ds(j * Q, Q)] = g * Q + xl
        sidx_f[pl.ds(j * Q, Q)] = g * Q + xf
        sidx_m[pl.ds(j * Q, Q)] = g * Q + xm
        didx[pl.ds(j * Q, Q)] = g * MAX_LEN + pv

    pltpu.sync_copy(mmap.at[pl.ds(b * Q, Q)], mrows)
    g_kl = pltpu.make_async_copy(ks.at[sidx_l], krl, sem_gkl)
    g_kf = pltpu.make_async_copy(ks.at[sidx_f], krf, sem_gkf)
    g_km = pltpu.make_async_copy(ks.at[sidx_m], krm, sem_gkm)
    g_vl = pltpu.make_async_copy(vs.at[sidx_l], vrl, sem_gvl)
    g_vf = pltpu.make_async_copy(vs.at[sidx_f], vrf, sem_gvf)
    g_vm = pltpu.make_async_copy(vs.at[sidx_m], vrm, sem_gvm)
    g_kl.start()
    g_kf.start()
    g_km.start()
    g_vl.start()
    g_vf.start()
    g_vm.start()
    g_kl.wait()
    g_kf.wait()
    g_km.wait()
    g_vl.wait()
    g_vf.wait()
    g_vm.wait()

    def _blend_body(t, carry):
        r = t // CHUNKS_PER_ROW
        ch = t % CHUNKS_PER_ROW
        q = r % Q
        off = ch * LANES
        m = mrows[q, pl.ds(off, LANES)]
        xk = jnp.where(m == 1, krm[r, pl.ds(off, LANES)], krf[r, pl.ds(off, LANES)])
        krl[r, pl.ds(off, LANES)] = jnp.where(m == 2, krl[r, pl.ds(off, LANES)], xk)
        xv = jnp.where(m == 1, vrm[r, pl.ds(off, LANES)], vrf[r, pl.ds(off, LANES)])
        vrl[r, pl.ds(off, LANES)] = jnp.where(m == 2, vrl[r, pl.ds(off, LANES)], xv)
        return carry

    lax.fori_loop(0, SCATTER_ROWS * CHUNKS_PER_ROW, _blend_body, 0)

    _copy_stream(kc, ko, row0, cb0, cb1, sem_r0, sem_r1, sem_w0, sem_w1)
    _copy_stream(vc, vo, row0, cb0, cb1, sem_r0, sem_r1, sem_w0, sem_w1)
    s_k = pltpu.make_async_copy(krl, ko.at[didx], sem_sk)
    s_v = pltpu.make_async_copy(vrl, vo.at[didx], sem_sv)
    s_k.start()
    s_v.start()
    s_k.wait()
    s_v.wait()


_PAIR_PAT = "".join(
    "2" if (d % 2) == (1 if d >= 64 else 0) else "0" for d in range(DH))
_TRIP_EVEN = (
    "2020201020202010202020102010201020202010202020102020201020102010"
    "2121210121212101212122220022022200020022002202220002022200220222")
_TRIP_ODD = (
    "2220220022202000222022002200200022202200222101210101012101210121"
    "0202021202020212020202120212021202020212021202120202021202120212")


def _member_map(pos):
    iota = jnp.arange(Q, dtype=jnp.int32)[None, :]
    nxt = jnp.concatenate([pos[:, 1:], jnp.full((B, 1), -1, jnp.int32)], axis=1)
    prv = jnp.concatenate([jnp.full((B, 1), -1, jnp.int32), pos[:, :-1]], axis=1)
    xl = jnp.where(pos != nxt, iota, Q - 1)
    xl = lax.rev(lax.cummin(lax.rev(xl, (1,)), axis=1), (1,))
    xf = jnp.where(pos != prv, iota, 0)
    xf = lax.cummax(xf, axis=1)
    size = (xl - xf + 1)[:, :, None]
    pair = jnp.asarray([int(ch) for ch in _PAIR_PAT], jnp.int32)
    trip_e = jnp.asarray([int(ch) for ch in _TRIP_EVEN], jnp.int32)
    trip_o = jnp.asarray([int(ch) for ch in _TRIP_ODD], jnp.int32)
    trip = jnp.where((xf % 2 == 0)[:, :, None], trip_e[None, None, :],
                     trip_o[None, None, :])
    m = jnp.where(size == 3, trip, pair[None, None, :])
    return jnp.where(size == 1, 0, m).astype(jnp.int32)


@jax.jit
def _sc_cache_update(ks_flat, vs_flat, pos_flat, mmap_flat, kc_flat, vc_flat):
    mesh = plsc.VectorSubcoreMesh(core_axis_name="c", subcore_axis_name="s")
    run = pl.kernel(
        _sc_body,
        out_type=[
            jax.ShapeDtypeStruct((TOTAL_ROWS, DH), jnp.float32),
            jax.ShapeDtypeStruct((TOTAL_ROWS, DH), jnp.float32),
        ],
        mesh=mesh,
        scratch_types=[
            pltpu.VMEM((Q,), jnp.int32),
            pltpu.VMEM((2 * Q,), jnp.int32),
            pltpu.VMEM((SCATTER_ROWS,), jnp.int32),
            pltpu.VMEM((SCATTER_ROWS,), jnp.int32),
            pltpu.VMEM((SCATTER_ROWS,), jnp.int32),
            pltpu.VMEM((SCATTER_ROWS,), jnp.int32),
            pltpu.VMEM((SCATTER_ROWS, DH), jnp.float32),
            pltpu.VMEM((SCATTER_ROWS, DH), jnp.float32),
            pltpu.VMEM((SCATTER_ROWS, DH), jnp.float32),
            pltpu.VMEM((SCATTER_ROWS, DH), jnp.float32),
            pltpu.VMEM((SCATTER_ROWS, DH), jnp.float32),
            pltpu.VMEM((SCATTER_ROWS, DH), jnp.float32),
            pltpu.VMEM((Q, DH), jnp.int32),
            pltpu.VMEM((COPY_CHUNK, DH), jnp.float32),
            pltpu.VMEM((COPY_CHUNK, DH), jnp.float32),
        ] + [pltpu.SemaphoreType.DMA] * 12,
    )
    return run(ks_flat, vs_flat, pos_flat, mmap_flat, kc_flat, vc_flat)


def kernel(key_states, value_states, position_ids, k_cache, v_cache, n_positions):
    pos = position_ids.astype(jnp.int32)
    ks_flat = key_states.reshape(B * H * Q, DH)
    vs_flat = value_states.reshape(B * H * Q, DH)
    mmap_flat = _member_map(pos).reshape(B * Q, DH)
    kc_flat = k_cache.reshape(TOTAL_ROWS, DH)
    vc_flat = v_cache.reshape(TOTAL_ROWS, DH)
    k_out, v_out = _sc_cache_update(ks_flat, vs_flat, pos.reshape(B * Q),
                                    mmap_flat, kc_flat, vc_flat)
    return (
        k_out.reshape(B, H, MAX_LEN, DH),
        v_out.reshape(B, H, MAX_LEN, DH),
    )


# speedup vs baseline: 1.2923x; 1.0013x over previous
import jax
import jax.numpy as jnp
from jax import lax
from jax.experimental import pallas as pl
from jax.experimental.pallas import tpu as pltpu
from jax.experimental.pallas import tpu_sc as plsc

B, H, Q, DH = 16, 8, 16, 128
MAX_LEN = 4096
N_POSITIONS = 2048

NUM_WORKERS = 32
TOTAL_ROWS = B * H * MAX_LEN
ROWS_PER_WORKER = TOTAL_ROWS // NUM_WORKERS
GROUPS_PER_WORKER = ROWS_PER_WORKER // MAX_LEN
SCATTER_ROWS = GROUPS_PER_WORKER * Q
LANES = 16
CHUNKS_PER_ROW = DH // LANES


COPY_CHUNK = 256
N_CHUNKS = ROWS_PER_WORKER // COPY_CHUNK


def _copy_stream(src, dst, row0, b0, b1, sr0, sr1, sw0, sw1):
    def rd(c, buf, sem):
        return pltpu.make_async_copy(
            src.at[pl.ds(row0 + c * COPY_CHUNK, COPY_CHUNK)], buf, sem)

    def wr(c, buf, sem):
        return pltpu.make_async_copy(
            buf, dst.at[pl.ds(row0 + c * COPY_CHUNK, COPY_CHUNK)], sem)

    rd(0, b0, sr0).start()

    def body(p, carry):
        c0 = 2 * p

        @pl.when(p > 0)
        def _():
            wr(c0 - 1, b1, sw1).wait()

        rd(c0 + 1, b1, sr1).start()
        rd(c0, b0, sr0).wait()
        wr(c0, b0, sw0).start()

        @pl.when(p < N_CHUNKS // 2 - 1)
        def _():
            wr(c0, b0, sw0).wait()
            rd(c0 + 2, b0, sr0).start()

        rd(c0 + 1, b1, sr1).wait()
        wr(c0 + 1, b1, sw1).start()
        return carry

    lax.fori_loop(0, N_CHUNKS // 2, body, 0)
    wr(N_CHUNKS - 2, b0, sw0).wait()
    wr(N_CHUNKS - 1, b1, sw1).wait()


def _sc_body(ks, vs, pos, mmap, kc, vc, ko, vo,
             pos_v, ext, sidx_l, sidx_f, sidx_m, didx,
             krl, krf, krm, vrl, vrf, vrm, mrows,
             cb0, cb1,
             sem_r0, sem_r1, sem_w0, sem_w1,
             sem_gkl, sem_gkf, sem_gkm, sem_gvl, sem_gvf, sem_gvm,
             sem_sk, sem_sv):
    c = lax.axis_index("c")
    s = lax.axis_index("s")
    wid = s * 2 + c
    row0 = wid * ROWS_PER_WORKER
    g0 = wid * GROUPS_PER_WORKER
    b = g0 // H

    pltpu.sync_copy(pos.at[pl.ds(b * Q, Q)], pos_v)
    pv = pos_v[...]
    iota = lax.iota(jnp.int32, Q)
    ext[pl.ds(Q, Q)] = jnp.full((Q,), -1, jnp.int32)
    ext[pl.ds(0, Q)] = pv
    nxt = ext[pl.ds(1, Q)]
    ext[pl.ds(0, Q)] = jnp.full((Q,), -1, jnp.int32)
    ext[pl.ds(Q, Q)] = pv
    prv = ext[pl.ds(Q - 1, Q)]
    xl = jnp.where(pv != nxt, iota, Q - 1)
    for k in (1, 2, 4, 8):
        ext[pl.ds(Q, Q)] = jnp.full((Q,), Q - 1, jnp.int32)
        ext[pl.ds(0, Q)] = xl
        xl = jnp.minimum(xl, ext[pl.ds(k, Q)])
    xf = jnp.where(pv != prv, iota, 0)
    for k in (1, 2, 4, 8):
        ext[pl.ds(0, Q)] = jnp.zeros((Q,), jnp.int32)
        ext[pl.ds(Q, Q)] = xf
        xf = jnp.maximum(xf, ext[pl.ds(Q - k, Q)])
    xm = jnp.minimum(xf + 1, xl)
    for j in range(GROUPS_PER_WORKER):
        g = g0 + j
        sidx_l[pl.ds(j * Q, Q)] = g * Q + xl
        sidx_f[pl.ds(j * Q, Q)] = g * Q + xf
        sidx_m[pl.ds(j * Q, Q)] = g * Q + xm
        didx[pl.ds(j * Q, Q)] = g * MAX_LEN + pv

    pltpu.sync_copy(mmap.at[pl.ds(b * Q, Q)], mrows)
    g_kl = pltpu.make_async_copy(ks.at[sidx_l], krl, sem_gkl)
    g_kf = pltpu.make_async_copy(ks.at[sidx_f], krf, sem_gkf)
    g_km = pltpu.make_async_copy(ks.at[sidx_m], krm, sem_gkm)
    g_vl = pltpu.make_async_copy(vs.at[sidx_l], vrl, sem_gvl)
    g_vf = pltpu.make_async_copy(vs.at[sidx_f], vrf, sem_gvf)
    g_vm = pltpu.make_async_copy(vs.at[sidx_m], vrm, sem_gvm)
    g_kl.start()
    g_kf.start()
    g_km.start()
    g_vl.start()
    g_vf.start()
    g_vm.start()
    g_kl.wait()
    g_kf.wait()
    g_km.wait()
    g_vl.wait()
    g_vf.wait()
    g_vm.wait()

    def _blend_body(t, carry):
        r = t // CHUNKS_PER_ROW
        ch = t % CHUNKS_PER_ROW
        q = r % Q
        off = ch * LANES
        m = mrows[q, pl.ds(off, LANES)]
        xk = jnp.where(m == 1, krm[r, pl.ds(off, LANES)], krf[r, pl.ds(off, LANES)])
        krl[r, pl.ds(off, LANES)] = jnp.where(m == 2, krl[r, pl.ds(off, LANES)], xk)
        xv = jnp.where(m == 1, vrm[r, pl.ds(off, LANES)], vrf[r, pl.ds(off, LANES)])
        vrl[r, pl.ds(off, LANES)] = jnp.where(m == 2, vrl[r, pl.ds(off, LANES)], xv)
        return carry

    lax.fori_loop(0, SCATTER_ROWS * CHUNKS_PER_ROW, _blend_body, 0)

    _copy_stream(kc, ko, row0, cb0, cb1, sem_r0, sem_r1, sem_w0, sem_w1)
    _copy_stream(vc, vo, row0, cb0, cb1, sem_r0, sem_r1, sem_w0, sem_w1)
    s_k = pltpu.make_async_copy(krl, ko.at[didx], sem_sk)
    s_v = pltpu.make_async_copy(vrl, vo.at[didx], sem_sv)
    s_k.start()
    s_v.start()
    s_k.wait()
    s_v.wait()


_PAIR_PAT = "".join(
    "2" if (d % 2) == (1 if d >= 64 else 0) else "0" for d in range(DH))
_TRIP_EVEN = (
    "2020201020202010202020102010201020202010202020102020201020102010"
    "2121210121212101212122220022022200020022002202220002022200220222")
_TRIP_ODD = (
    "2220220022202000222022002200200022202200222101210101012101210121"
    "0202021202020212020202120212021202020212021202120202021202120212")


def _member_map(pos):
    iota = jnp.arange(Q, dtype=jnp.int32)[None, :]
    nxt = jnp.concatenate([pos[:, 1:], jnp.full((B, 1), -1, jnp.int32)], axis=1)
    prv = jnp.concatenate([jnp.full((B, 1), -1, jnp.int32), pos[:, :-1]], axis=1)
    xl = jnp.where(pos != nxt, iota, Q - 1)
    xl = lax.rev(lax.cummin(lax.rev(xl, (1,)), axis=1), (1,))
    xf = jnp.where(pos != prv, iota, 0)
    xf = lax.cummax(xf, axis=1)
    size = (xl - xf + 1)[:, :, None]
    pair = jnp.asarray([int(ch) for ch in _PAIR_PAT], jnp.int32)
    trip_e = jnp.asarray([int(ch) for ch in _TRIP_EVEN], jnp.int32)
    trip_o = jnp.asarray([int(ch) for ch in _TRIP_ODD], jnp.int32)
    trip = jnp.where((xf % 2 == 0)[:, :, None], trip_e[None, None, :],
                     trip_o[None, None, :])
    m = jnp.where(size == 3, trip, pair[None, None, :])
    return jnp.where(size == 1, 0, m).astype(jnp.int32)


@jax.jit
def _sc_cache_update(ks_flat, vs_flat, pos_flat, mmap_flat, kc_flat, vc_flat):
    mesh = plsc.VectorSubcoreMesh(core_axis_name="c", subcore_axis_name="s")
    run = pl.kernel(
        _sc_body,
        out_type=[
            jax.ShapeDtypeStruct((TOTAL_ROWS, DH), jnp.float32),
            jax.ShapeDtypeStruct((TOTAL_ROWS, DH), jnp.float32),
        ],
        mesh=mesh,
        scratch_types=[
            pltpu.VMEM((Q,), jnp.int32),
            pltpu.VMEM((2 * Q,), jnp.int32),
            pltpu.VMEM((SCATTER_ROWS,), jnp.int32),
            pltpu.VMEM((SCATTER_ROWS,), jnp.int32),
            pltpu.VMEM((SCATTER_ROWS,), jnp.int32),
            pltpu.VMEM((SCATTER_ROWS,), jnp.int32),
            pltpu.VMEM((SCATTER_ROWS, DH), jnp.float32),
            pltpu.VMEM((SCATTER_ROWS, DH), jnp.float32),
            pltpu.VMEM((SCATTER_ROWS, DH), jnp.float32),
            pltpu.VMEM((SCATTER_ROWS, DH), jnp.float32),
            pltpu.VMEM((SCATTER_ROWS, DH), jnp.float32),
            pltpu.VMEM((SCATTER_ROWS, DH), jnp.float32),
            pltpu.VMEM((Q, DH), jnp.int32),
            pltpu.VMEM((COPY_CHUNK, DH), jnp.float32),
            pltpu.VMEM((COPY_CHUNK, DH), jnp.float32),
        ] + [pltpu.SemaphoreType.DMA] * 12,
    )
    return run(ks_flat, vs_flat, pos_flat, mmap_flat, kc_flat, vc_flat)


def kernel(key_states, value_states, position_ids, k_cache, v_cache, n_positions):
    pos = position_ids.astype(jnp.int32)
    ks_flat = key_states.reshape(B * H * Q, DH)
    vs_flat = value_states.reshape(B * H * Q, DH)
    mmap_flat = _member_map(pos).reshape(B * Q, DH)
    kc_flat = k_cache.reshape(TOTAL_ROWS, DH)
    vc_flat = v_cache.reshape(TOTAL_ROWS, DH)
    k_out, v_out = _sc_cache_update(ks_flat, vs_flat, pos.reshape(B * Q),
                                    mmap_flat, kc_flat, vc_flat)
    return (
        k_out.reshape(B, H, MAX_LEN, DH),
        v_out.reshape(B, H, MAX_LEN, DH),
    )
